# baseline (speedup 1.0000x reference)
import jax
import jax.numpy as jnp
from jax.experimental import pallas as pl

S, K = 1024, 32


def _copy_kernel(x_ref, o_ref):
    o_ref[...] = x_ref[...]


def _index_points(points, idx):
    return jax.vmap(lambda p, i: p[i])(points, idx)


def _fps(xyz, n_samples):
    b, n, _ = xyz.shape
    idxs0 = jnp.zeros((b, n_samples), dtype=jnp.int32)
    dists0 = jnp.full((b, n), 1e10, dtype=jnp.float32)
    farthest0 = jnp.zeros((b,), dtype=jnp.int32)

    def body(i, state):
        idxs, dists, farthest = state
        idxs = idxs.at[:, i].set(farthest)
        centroid = jnp.take_along_axis(xyz, farthest[:, None, None], axis=1)
        d = jnp.sum((xyz - centroid) ** 2, axis=-1)
        dists = jnp.minimum(dists, d)
        farthest = jnp.argmax(dists, axis=-1).astype(jnp.int32)
        return (idxs, dists, farthest)

    idxs, _, _ = jax.lax.fori_loop(0, n_samples, body, (idxs0, dists0, farthest0))
    return idxs


def kernel(xyz, points, affine_alpha_first, affine_beta_first):
    b = xyz.shape[0]
    xyz = pl.pallas_call(
        _copy_kernel,
        out_shape=jax.ShapeDtypeStruct(xyz.shape, xyz.dtype),
    )(xyz)
    fps_idx = _fps(xyz, S)
    new_xyz = _index_points(xyz, fps_idx)
    new_points = _index_points(points, fps_idx)
    sqr = -2.0 * jnp.einsum('bsc,bnc->bsn', new_xyz, xyz)
    sqr = sqr + jnp.sum(new_xyz ** 2, axis=-1)[:, :, None]
    sqr = sqr + jnp.sum(xyz ** 2, axis=-1)[:, None, :]
    _, idx = jax.lax.top_k(-sqr, K)
    grouped_points = _index_points(points, idx)
    mean = new_points[:, :, None, :]
    std = jnp.std(grouped_points - mean, axis=-2, keepdims=True, ddof=1)
    grouped_points = (grouped_points - mean) / (std + 1e-05)
    grouped_points = affine_alpha_first * grouped_points + affine_beta_first
    rep = jnp.broadcast_to(new_points.reshape(b, S, 1, -1), (b, S, K, new_points.shape[-1]))
    out = jnp.concatenate([grouped_points, rep], axis=-1)
    return (new_xyz, out)


# speedup vs baseline: 1.8717x; 1.8717x over previous
import jax
import jax.numpy as jnp
from jax.experimental import pallas as pl

B, N, S, K, CH = 8, 4096, 1024, 32, 64


def _fps_body(xp_ref, idx_ref, nxyz_ref):
    x = xp_ref[0]
    y = xp_ref[1]
    z = xp_ref[2]
    col_n = jax.lax.broadcasted_iota(jnp.int32, (B, N), 1)
    col_s = jax.lax.broadcasted_iota(jnp.int32, (B, S), 1)

    dists0 = x * 0.0 + 1e10
    zero_s = x[:, :S] * 0.0
    far0 = jnp.zeros((B, 1), dtype=jnp.int32)
    idxs0 = zero_s.astype(jnp.int32)
    nx0 = zero_s
    ny0 = zero_s
    nz0 = zero_s

    def body(i, state):
        dists, far, idxs, nx, ny, nz = state
        sel = col_n == far
        cx = jnp.sum(jnp.where(sel, x, 0.0), axis=1, keepdims=True)
        cy = jnp.sum(jnp.where(sel, y, 0.0), axis=1, keepdims=True)
        cz = jnp.sum(jnp.where(sel, z, 0.0), axis=1, keepdims=True)
        here_i = (col_s == i).astype(jnp.int32)
        here_f = here_i.astype(jnp.float32)
        idxs = idxs + here_i * far
        nx = nx + here_f * cx
        ny = ny + here_f * cy
        nz = nz + here_f * cz
        dx = x - cx
        dy = y - cy
        dz = z - cz
        d = (dx * dx + dy * dy) + dz * dz
        dists = jnp.minimum(dists, d)
        m = jnp.max(dists, axis=1, keepdims=True)
        far = jnp.min(jnp.where(dists == m, col_n, jnp.int32(2**30)),
                      axis=1, keepdims=True)
        return (dists, far, idxs, nx, ny, nz)

    state = jax.lax.fori_loop(0, S, body,
                              (dists0, far0, idxs0, nx0, ny0, nz0))
    _, _, idxs, nx, ny, nz = state
    idx_ref[...] = idxs
    nxyz_ref[0] = nx
    nxyz_ref[1] = ny
    nxyz_ref[2] = nz


def _fps_pallas(xp):
    return pl.pallas_call(
        _fps_body,
        out_shape=(
            jax.ShapeDtypeStruct((B, S), jnp.int32),
            jax.ShapeDtypeStruct((3, B, S), jnp.float32),
        ),
    )(xp)


def _index_points(points, idx):
    return jax.vmap(lambda p, i: p[i])(points, idx)


def kernel(xyz, points, affine_alpha_first, affine_beta_first):
    xp = jnp.transpose(xyz, (2, 0, 1))
    fps_idx, nxyz = _fps_pallas(xp)
    new_xyz = jnp.transpose(nxyz, (1, 2, 0))

    new_points = _index_points(points, fps_idx)
    sqr = -2.0 * jnp.einsum('bsc,bnc->bsn', new_xyz, xyz)
    sqr = sqr + jnp.sum(new_xyz ** 2, axis=-1)[:, :, None]
    sqr = sqr + jnp.sum(xyz ** 2, axis=-1)[:, None, :]
    _, idx = jax.lax.top_k(-sqr, K)
    grouped_points = _index_points(points, idx)
    mean = new_points[:, :, None, :]
    std = jnp.std(grouped_points - mean, axis=-2, keepdims=True, ddof=1)
    grouped_points = (grouped_points - mean) / (std + 1e-05)
    grouped_points = affine_alpha_first * grouped_points + affine_beta_first
    rep = jnp.broadcast_to(new_points.reshape(B, S, 1, -1), (B, S, K, CH))
    out = jnp.concatenate([grouped_points, rep], axis=-1)
    return (new_xyz, out)


# speedup vs baseline: 3.0717x; 1.6411x over previous
import jax
import jax.numpy as jnp
from jax.experimental import pallas as pl

B, N, S, K, CH = 8, 4096, 1024, 32, 64


def _fps_body(xp_ref, idx_ref, nxyz_ref):
    x = xp_ref[0]
    y = xp_ref[1]
    z = xp_ref[2]
    col_n = jax.lax.broadcasted_iota(jnp.int32, (B, N), 1)
    col_s = jax.lax.broadcasted_iota(jnp.int32, (B, S), 1)

    dists0 = x * 0.0 + 1e10
    zero_s = x[:, :S] * 0.0
    far0 = jnp.zeros((B, 1), dtype=jnp.int32)
    idxs0 = zero_s.astype(jnp.int32)
    nx0 = zero_s
    ny0 = zero_s
    nz0 = zero_s

    def body(i, state):
        dists, far, idxs, nx, ny, nz = state
        sel = col_n == far
        cx = jnp.sum(jnp.where(sel, x, 0.0), axis=1, keepdims=True)
        cy = jnp.sum(jnp.where(sel, y, 0.0), axis=1, keepdims=True)
        cz = jnp.sum(jnp.where(sel, z, 0.0), axis=1, keepdims=True)
        here_i = (col_s == i).astype(jnp.int32)
        here_f = here_i.astype(jnp.float32)
        idxs = idxs + here_i * far
        nx = nx + here_f * cx
        ny = ny + here_f * cy
        nz = nz + here_f * cz
        dx = x - cx
        dy = y - cy
        dz = z - cz
        d = (dx * dx + dy * dy) + dz * dz
        dists = jnp.minimum(dists, d)
        m = jnp.max(dists, axis=1, keepdims=True)
        far = jnp.min(jnp.where(dists == m, col_n, jnp.int32(2**30)),
                      axis=1, keepdims=True)
        return (dists, far, idxs, nx, ny, nz)

    state = jax.lax.fori_loop(0, S, body,
                              (dists0, far0, idxs0, nx0, ny0, nz0))
    _, _, idxs, nx, ny, nz = state
    idx_ref[...] = idxs
    nxyz_ref[0] = nx
    nxyz_ref[1] = ny
    nxyz_ref[2] = nz


def _fps_pallas(xp):
    return pl.pallas_call(
        _fps_body,
        out_shape=(
            jax.ShapeDtypeStruct((B, S), jnp.int32),
            jax.ShapeDtypeStruct((3, B, S), jnp.float32),
        ),
    )(xp)


TS = 128


def _knn_body(q_ref, xt_ref, idx_ref):
    q = q_ref[0]
    xt = xt_ref[0]
    x0 = xt[0:1]
    x1 = xt[1:2]
    x2 = xt[2:3]
    xn = (x0 * x0 + x1 * x1) + x2 * x2
    qx = jax.lax.dot_general(q, xt, (((1,), (0,)), ((), ())),
                             preferred_element_type=jnp.float32)
    d = xn - 2.0 * qx
    col = jax.lax.broadcasted_iota(jnp.int32, (TS, N), 1)
    col_out = jax.lax.broadcasted_iota(jnp.int32, (TS, 128), 1)
    out0 = (d[:, :128] * 0.0).astype(jnp.int32)

    def body(j, state):
        d, out = state
        m = jnp.min(d, axis=1, keepdims=True)
        eq = d == m
        chosen = jnp.min(jnp.where(eq, col, jnp.int32(2**30)),
                         axis=1, keepdims=True)
        out = out + (col_out == j).astype(jnp.int32) * chosen
        d = jnp.where(eq & (col == chosen), jnp.float32(jnp.inf), d)
        return (d, out)

    _, out = jax.lax.fori_loop(0, K, body, (d, out0))
    idx_ref[0] = out


def _knn_pallas(q8, xt8):
    return pl.pallas_call(
        _knn_body,
        grid=(B, S // TS),
        in_specs=[
            pl.BlockSpec((1, TS, 8), lambda b, j: (b, j, 0)),
            pl.BlockSpec((1, 8, N), lambda b, j: (b, 0, 0)),
        ],
        out_specs=pl.BlockSpec((1, TS, 128), lambda b, j: (b, j, 0)),
        out_shape=jax.ShapeDtypeStruct((B, S, 128), jnp.int32),
    )(q8, xt8)


def _index_points(points, idx):
    return jax.vmap(lambda p, i: p[i])(points, idx)


def kernel(xyz, points, affine_alpha_first, affine_beta_first):
    xp = jnp.transpose(xyz, (2, 0, 1))
    fps_idx, nxyz = _fps_pallas(xp)
    new_xyz = jnp.transpose(nxyz, (1, 2, 0))

    q8 = jnp.concatenate(
        [jnp.transpose(nxyz, (1, 2, 0)),
         jnp.zeros((B, S, 5), jnp.float32)], axis=-1)
    xt8 = jnp.concatenate(
        [xp.transpose(1, 0, 2), jnp.zeros((B, 5, N), jnp.float32)],
        axis=1)
    idx = _knn_pallas(q8, xt8)[:, :, :K]

    new_points = _index_points(points, fps_idx)
    grouped_points = _index_points(points, idx)
    mean = new_points[:, :, None, :]
    std = jnp.std(grouped_points - mean, axis=-2, keepdims=True, ddof=1)
    grouped_points = (grouped_points - mean) / (std + 1e-05)
    grouped_points = affine_alpha_first * grouped_points + affine_beta_first
    rep = jnp.broadcast_to(new_points.reshape(B, S, 1, -1), (B, S, K, CH))
    out = jnp.concatenate([grouped_points, rep], axis=-1)
    return (new_xyz, out)


# speedup vs baseline: 5.5608x; 1.8103x over previous
import functools

import jax
import jax.numpy as jnp
from jax import lax
from jax.experimental import pallas as pl
from jax.experimental.pallas import tpu as pltpu
from jax.experimental.pallas import tpu_sc as plsc

B, N, S, K, CH = 8, 4096, 1024, 32, 64


def _fps_body(xp_ref, idx_ref, nxyz_ref):
    x = xp_ref[0]
    y = xp_ref[1]
    z = xp_ref[2]
    col_n = jax.lax.broadcasted_iota(jnp.int32, (B, N), 1)
    col_s = jax.lax.broadcasted_iota(jnp.int32, (B, S), 1)

    dists0 = x * 0.0 + 1e10
    zero_s = x[:, :S] * 0.0
    far0 = jnp.zeros((B, 1), dtype=jnp.int32)
    idxs0 = zero_s.astype(jnp.int32)
    nx0 = zero_s
    ny0 = zero_s
    nz0 = zero_s

    def body(i, state):
        dists, far, idxs, nx, ny, nz = state
        sel = col_n == far
        cx = jnp.sum(jnp.where(sel, x, 0.0), axis=1, keepdims=True)
        cy = jnp.sum(jnp.where(sel, y, 0.0), axis=1, keepdims=True)
        cz = jnp.sum(jnp.where(sel, z, 0.0), axis=1, keepdims=True)
        here_i = (col_s == i).astype(jnp.int32)
        here_f = here_i.astype(jnp.float32)
        idxs = idxs + here_i * far
        nx = nx + here_f * cx
        ny = ny + here_f * cy
        nz = nz + here_f * cz
        dx = x - cx
        dy = y - cy
        dz = z - cz
        d = (dx * dx + dy * dy) + dz * dz
        dists = jnp.minimum(dists, d)
        m = jnp.max(dists, axis=1, keepdims=True)
        far = jnp.min(jnp.where(dists == m, col_n, jnp.int32(2**30)),
                      axis=1, keepdims=True)
        return (dists, far, idxs, nx, ny, nz)

    state = jax.lax.fori_loop(0, S, body,
                              (dists0, far0, idxs0, nx0, ny0, nz0))
    _, _, idxs, nx, ny, nz = state
    row_b = jax.lax.broadcasted_iota(jnp.int32, (B, S), 0)
    idx_ref[...] = idxs + row_b * N
    nxyz_ref[0] = nx
    nxyz_ref[1] = ny
    nxyz_ref[2] = nz


def _fps_pallas(xp):
    return pl.pallas_call(
        _fps_body,
        out_shape=(
            jax.ShapeDtypeStruct((B, S), jnp.int32),
            jax.ShapeDtypeStruct((3, B, S), jnp.float32),
        ),
    )(xp)


TS = 128


def _knn_body(q_ref, xt_ref, idx_ref):
    q = q_ref[0]
    xt = xt_ref[0]
    x0 = xt[0:1]
    x1 = xt[1:2]
    x2 = xt[2:3]
    xn = (x0 * x0 + x1 * x1) + x2 * x2
    qx = jax.lax.dot_general(q, xt, (((1,), (0,)), ((), ())),
                             preferred_element_type=jnp.float32)
    d = xn - 2.0 * qx
    col = jax.lax.broadcasted_iota(jnp.int32, (TS, N), 1)
    col_out = jax.lax.broadcasted_iota(jnp.int32, (TS, 128), 1)
    out0 = (d[:, :128] * 0.0).astype(jnp.int32)
    nbase = pl.program_id(0) * N

    def body(j, state):
        d, out = state
        m = jnp.min(d, axis=1, keepdims=True)
        eq = d == m
        chosen = jnp.min(jnp.where(eq, col, jnp.int32(2**30)),
                         axis=1, keepdims=True)
        out = out + (col_out == j).astype(jnp.int32) * (chosen + nbase)
        d = jnp.where(eq & (col == chosen), jnp.float32(jnp.inf), d)
        return (d, out)

    _, out = jax.lax.fori_loop(0, K, body, (d, out0))
    idx_ref[0] = out


def _knn_pallas(q8, xt8):
    return pl.pallas_call(
        _knn_body,
        grid=(B, S // TS),
        in_specs=[
            pl.BlockSpec((1, TS, 8), lambda b, j: (b, j, 0)),
            pl.BlockSpec((1, 8, N), lambda b, j: (b, 0, 0)),
        ],
        out_specs=pl.BlockSpec((1, TS, 128), lambda b, j: (b, j, 0)),
        out_shape=jax.ShapeDtypeStruct((B, S, 128), jnp.int32),
    )(q8, xt8)


ROWS = B * S
NW = 32
RPW = ROWS // NW
IW = 40
NC = CH // 16


def _sc_group_body(points_hbm, idx_hbm, alpha_hbm, beta_hbm, out_hbm,
                   idx_v, gbuf, obuf, av, bv, sem):
    cid = lax.axis_index("c")
    sid = lax.axis_index("s")
    wid = sid * 2 + cid
    base = wid * RPW
    pltpu.sync_copy(idx_hbm.at[pl.ds(base, RPW)], idx_v)
    pltpu.sync_copy(alpha_hbm, av)
    pltpu.sync_copy(beta_hbm, bv)

    inv_k = jnp.float32(1.0 / K)
    inv_km1 = jnp.float32(1.0 / (K - 1))

    def row_body(r, carry):
        pltpu.async_copy(points_hbm.at[idx_v.at[r]], gbuf, sem).wait()
        m = [gbuf[0, pl.ds(16 * c, 16)] for c in range(NC)]
        s = [m[c] * 0.0 for c in range(NC)]
        q = [m[c] * 0.0 for c in range(NC)]
        for k in range(K):
            for c in range(NC):
                g = gbuf[1 + k, pl.ds(16 * c, 16)]
                s[c] = s[c] + g
                q[c] = q[c] + g * g
        sa = []
        for c in range(NC):
            var = (q[c] - s[c] * s[c] * inv_k) * inv_km1
            var = jnp.maximum(var, jnp.float32(1e-30))
            i = lax.bitcast_convert_type(var, jnp.int32)
            y = lax.bitcast_convert_type(
                jnp.int32(0x5F3759DF) - lax.shift_right_logical(i, 1),
                jnp.float32)
            for _ in range(3):
                y = y * (1.5 - 0.5 * var * y * y)
            std = var * y
            scale = 1.0 / (std + 1e-5)
            sa.append(scale * av[pl.ds(16 * c, 16)])
        for k in range(K):
            for c in range(NC):
                g = gbuf[1 + k, pl.ds(16 * c, 16)]
                obuf[0, k, pl.ds(16 * c, 16)] = \
                    (g - m[c]) * sa[c] + bv[pl.ds(16 * c, 16)]
                obuf[0, k, pl.ds(CH + 16 * c, 16)] = m[c]
        pltpu.sync_copy(obuf, out_hbm.at[pl.ds(base + r, 1)])
        return carry

    lax.fori_loop(0, RPW, row_body, jnp.int32(0))


def _sc_group(points_flat, idx40, alpha_v, beta_v):
    fn = pl.kernel(
        _sc_group_body,
        out_type=jax.ShapeDtypeStruct((ROWS, K, 2 * CH), jnp.float32),
        mesh=plsc.VectorSubcoreMesh(core_axis_name="c", subcore_axis_name="s"),
        scratch_types=[
            pltpu.VMEM((RPW, IW), jnp.int32),
            pltpu.VMEM((IW, CH), jnp.float32),
            pltpu.VMEM((1, K, 2 * CH), jnp.float32),
            pltpu.VMEM((CH,), jnp.float32),
            pltpu.VMEM((CH,), jnp.float32),
            pltpu.SemaphoreType.DMA,
        ],
        compiler_params=pltpu.CompilerParams(use_tc_tiling_on_sc=False),
    )
    return fn(points_flat, idx40, alpha_v, beta_v)


def kernel(xyz, points, affine_alpha_first, affine_beta_first):
    xp = jnp.transpose(xyz, (2, 0, 1))
    fps_idx, nxyz = _fps_pallas(xp)
    new_xyz = jnp.transpose(nxyz, (1, 2, 0))

    q8 = jnp.concatenate(
        [jnp.transpose(nxyz, (1, 2, 0)),
         jnp.zeros((B, S, 5), jnp.float32)], axis=-1)
    xt8 = jnp.concatenate(
        [xp.transpose(1, 0, 2), jnp.zeros((B, 5, N), jnp.float32)],
        axis=1)
    idx = _knn_pallas(q8, xt8)[:, :, :K]

    idx40 = jnp.concatenate(
        [fps_idx.reshape(ROWS, 1), idx.reshape(ROWS, K),
         jnp.zeros((ROWS, IW - 1 - K), jnp.int32)], axis=1)
    points_flat = points.reshape(B * N, CH)
    alpha_v = affine_alpha_first.reshape(CH)
    beta_v = affine_beta_first.reshape(CH)
    out = _sc_group(points_flat, idx40, alpha_v, beta_v)
    return (new_xyz, out.reshape(B, S, K, 2 * CH))
